# baseline (speedup 1.0000x reference)
import functools

import jax
import jax.numpy as jnp
from jax import lax
from jax.experimental import pallas as pl
from jax.experimental.pallas import tpu as pltpu
from jax.experimental.pallas import tpu_sc as plsc

_TAU = 0.5
_ALPHA = 0.5
_LAM = 0.5

_BN = 20000


def _tc_body(q_ref, hqb_ref, haqb_ref, h_ref, ha_ref, out_ref, acc_h, acc_a):
    g = pl.program_id(0)
    k = pl.num_programs(0)
    row = q_ref[0] % 8
    hq = hqb_ref[pl.ds(row, 1), :]
    haq = haqb_ref[pl.ds(row, 1), :]

    inv_aq = jnp.minimum(lax.rsqrt(jnp.sum(hq * hq)), 1e8)
    inv_aaq = jnp.minimum(lax.rsqrt(jnp.sum(haq * haq)), 1e8)
    ri = lax.broadcasted_iota(jnp.int32, (8, 1), 0)
    scale = jnp.where(ri == 0, inv_aq, jnp.where(ri == 1, inv_aaq, 0.0)) / _TAU

    w8 = jnp.concatenate([hq, haq, jnp.zeros((6, 128), jnp.float32)], axis=0)
    ones1 = jnp.ones((1, 128), jnp.float32)
    dn = (((1,), (1,)), ((), ()))

    @pl.when(g == 0)
    def _():
        acc_h[...] = jnp.zeros_like(acc_h)
        acc_a[...] = jnp.zeros_like(acc_a)

    def accum(blk, acc):
        d = lax.dot_general(w8, blk, dn, preferred_element_type=jnp.float32)
        nsq = lax.dot_general(ones1, blk * blk, dn,
                              preferred_element_type=jnp.float32)
        inv_n = jnp.minimum(lax.rsqrt(nsq), 1e8)
        e = jnp.exp(d * inv_n * scale)
        acc[...] += jnp.sum(e, axis=1, keepdims=True)

    accum(h_ref[...], acc_h)
    accum(ha_ref[...], acc_a)

    @pl.when(g == k - 1)
    def _():
        out_ref[:, 0:1] = acc_h[...]
        out_ref[:, 1:2] = acc_a[...]


def _tc_sums(h, h_aug, q1):
    n = h.shape[0]
    grid = n // _BN
    grid_spec = pltpu.PrefetchScalarGridSpec(
        num_scalar_prefetch=1,
        grid=(grid,),
        in_specs=[
            pl.BlockSpec((8, 128), lambda g, qr: (qr[0] // 8, 0)),
            pl.BlockSpec((8, 128), lambda g, qr: (qr[0] // 8, 0)),
            pl.BlockSpec((_BN, 128), lambda g, qr: (g, 0)),
            pl.BlockSpec((_BN, 128), lambda g, qr: (g, 0)),
        ],
        out_specs=pl.BlockSpec((8, 2), lambda g, qr: (0, 0)),
        scratch_shapes=[
            pltpu.VMEM((8, 1), jnp.float32),
            pltpu.VMEM((8, 1), jnp.float32),
        ],
    )
    return pl.pallas_call(
        _tc_body,
        grid_spec=grid_spec,
        out_shape=jax.ShapeDtypeStruct((8, 2), jnp.float32),
    )(q1, h, h_aug, h, h_aug)


def _sum16(x):
    li = lax.broadcasted_iota(jnp.int32, (16,), 0)
    for sh in (8, 4, 2, 1):
        x = x + jnp.take(x, jnp.bitwise_xor(li, sh))
    return x


def _nrsqrt16(x):
    x = jnp.maximum(x, 1e-30)
    i = lax.bitcast_convert_type(x, jnp.int32)
    i = 0x5F3759DF - lax.shift_right_arithmetic(i, 1)
    y = lax.bitcast_convert_type(i, jnp.float32)
    for _ in range(4):
        y = y * (1.5 - 0.5 * x * y * y)
    return jnp.minimum(y, 1e8)


def _sc_pos(h, h_aug, pos, q16):
    n = h.shape[0]
    p = pos.shape[0]
    nc, ns = 1, 16
    rpt = p // ns
    mesh = plsc.VectorSubcoreMesh(core_axis_name="c", subcore_axis_name="s",
                                  num_cores=nc, num_subcores=ns)

    @functools.partial(
        pl.kernel,
        out_type=jax.ShapeDtypeStruct((nc, 16), jnp.float32),
        mesh=mesh,
        scratch_types=[
            pltpu.VMEM((rpt,), jnp.int32),
            pltpu.VMEM((rpt,), jnp.int32),
            pltpu.VMEM((rpt, 128), jnp.float32),
            pltpu.VMEM((rpt, 128), jnp.float32),
            pltpu.VMEM((1, 128), jnp.float32),
            pltpu.VMEM((1, 128), jnp.float32),
            pltpu.VMEM((rpt,), jnp.int32),
            pltpu.VMEM((1, 16), jnp.float32),
            pltpu.VMEM((16, 16), jnp.float32),
            pltpu.VMEM((1, 16), jnp.int32),
            pltpu.VMEM_SHARED((n,), jnp.int32),
            pltpu.VMEM_SHARED((16, 16), jnp.float32),
            pltpu.SemaphoreType.DMA,
            pltpu.SemaphoreType.DMA,
        ],
    )
    def k(h_hbm, ha_hbm, pos_hbm, q_hbm, out_hbm,
          idx_v, gid_v, rows_h, rows_a, hq_v, haq_v, t_v, part_v, tab_v,
          q_v, winner_sh, tab_sh, sem, sem2):
        c = lax.axis_index("c")
        s = lax.axis_index("s")
        base = s * rpt
        li = lax.broadcasted_iota(jnp.int32, (16,), 0)

        pltpu.sync_copy(pos_hbm.at[pl.ds(base, rpt)], idx_v)
        for g2 in range(rpt // 16):
            gid_v[pl.ds(16 * g2, 16)] = li + (base + 16 * g2)

        cp_h = pltpu.async_copy(h_hbm.at[idx_v], rows_h, sem)
        cp_a = pltpu.async_copy(ha_hbm.at[idx_v], rows_a, sem2)

        pltpu.sync_copy(gid_v, winner_sh.at[idx_v])
        pltpu.sync_copy(q_hbm, q_v)
        qvec = q_v[0, :]
        qs = qvec[0]
        pltpu.sync_copy(h_hbm.at[pl.ds(qs, 1)], hq_v)
        pltpu.sync_copy(ha_hbm.at[pl.ds(qs, 1)], haq_v)
        plsc.subcore_barrier()
        pltpu.sync_copy(winner_sh.at[idx_v], t_v)
        cp_h.wait()
        cp_a.wait()

        accq = jnp.zeros((16,), jnp.float32)
        accqa = jnp.zeros((16,), jnp.float32)
        accx = jnp.zeros((16,), jnp.float32)
        for ch in range(8):
            a = hq_v[0, pl.ds(16 * ch, 16)]
            b = haq_v[0, pl.ds(16 * ch, 16)]
            accq += a * a
            accqa += b * b
            accx += a * b
        inv_aq = _nrsqrt16(_sum16(accq))
        inv_aaq = _nrsqrt16(_sum16(accqa))
        cq_v = _sum16(accx) * inv_aq * inv_aaq

        m1 = jnp.zeros((16,), jnp.float32)
        ma1 = jnp.zeros((16,), jnp.float32)
        m2 = jnp.zeros((16,), jnp.float32)
        ma2 = jnp.zeros((16,), jnp.float32)
        cntv = jnp.zeros((16,), jnp.float32)
        hqc = [hq_v[0, pl.ds(16 * ch, 16)] for ch in range(8)]
        haqc = [haq_v[0, pl.ds(16 * ch, 16)] for ch in range(8)]

        for g2 in range(rpt // 16):
            def row_body(i, carry, _g2=g2):
                d1v, da2v, nnv, d2v, da1v, nav = carry
                r = 16 * _g2 + i
                a1 = a2 = nn = b1 = b2 = bb = jnp.zeros((16,), jnp.float32)
                for ch in range(8):
                    xh = rows_h[r, pl.ds(16 * ch, 16)]
                    xa = rows_a[r, pl.ds(16 * ch, 16)]
                    a1 += xh * hqc[ch]
                    a2 += xh * haqc[ch]
                    nn += xh * xh
                    b1 += xa * hqc[ch]
                    b2 += xa * haqc[ch]
                    bb += xa * xa
                sel = li == i
                return (jnp.where(sel, _sum16(a1), d1v),
                        jnp.where(sel, _sum16(a2), da2v),
                        jnp.where(sel, _sum16(nn), nnv),
                        jnp.where(sel, _sum16(b1), d2v),
                        jnp.where(sel, _sum16(b2), da1v),
                        jnp.where(sel, _sum16(bb), nav))

            z6 = tuple(jnp.zeros((16,), jnp.float32) for _ in range(6))
            d1v, da2v, nnv, d2v, da1v, nav = lax.fori_loop(0, 16, row_body, z6)
            inv_h = _nrsqrt16(nnv)
            inv_a = _nrsqrt16(nav)
            tt = t_v[pl.ds(16 * g2, 16)]
            pv = idx_v[pl.ds(16 * g2, 16)]
            gids = li + 16 * g2 + base
            u = jnp.where((tt == gids) & (pv != qvec), 1.0, 0.0)
            m1 += u * (d1v * inv_h * inv_aq)
            ma2 += u * (da2v * inv_h * inv_aaq)
            m2 += u * (d2v * inv_a * inv_aq)
            ma1 += u * (da1v * inv_a * inv_aaq)
            cntv += u

        tile0 = jnp.where(s == 0, 1.0, 0.0)
        pvec = (jnp.where(li == 0, _sum16(m1), 0.0)
                + jnp.where(li == 1, _sum16(ma1), 0.0)
                + jnp.where(li == 2, _sum16(m2), 0.0)
                + jnp.where(li == 3, _sum16(ma2), 0.0)
                + jnp.where(li == 4, _sum16(cntv), 0.0)
                + jnp.where(li == 5, cq_v * tile0, 0.0))
        part_v[0, :] = pvec
        pltpu.sync_copy(part_v, tab_sh.at[pl.ds(s, 1)])
        plsc.subcore_barrier()

        @pl.when(s == 0)
        def _():
            pltpu.sync_copy(tab_sh, tab_v)
            acc = tab_v[0, :]
            for r in range(1, 16):
                acc = acc + tab_v[r, :]
            part_v[0, :] = acc
            pltpu.sync_copy(part_v, out_hbm.at[pl.ds(c, 1)])

    return k(h, h_aug, pos, q16)


def kernel(h, h_aug, q, pos, edge_index):
    n, d = h.shape
    qi = jnp.asarray(q, jnp.int32)
    q1 = jnp.full((1,), qi, jnp.int32)
    q16 = jnp.full((1, 16), qi, jnp.int32)

    sc = _sc_pos(h, h_aug, pos, q16)
    sums = _tc_sums(h, h_aug, q1)

    li = jnp.arange(16)
    wm = (0.5 * ((li == 0) | (li == 1)) + 0.25 * ((li == 2) | (li == 3))
          ).astype(jnp.float32)
    r = sc[0]
    dotm = jnp.sum(r * wm)
    cnt = jnp.sum(r * (li == 4))
    cq = jnp.sum(r * (li == 5))
    log_term = 0.5 * jnp.sum(jnp.log(sums[0:2, :]))
    return log_term - dotm / (_TAU * cnt) - 0.5 * cq / _TAU


# speedup vs baseline: 1.0691x; 1.0691x over previous
import functools

import jax
import jax.numpy as jnp
from jax import lax
from jax.experimental import pallas as pl
from jax.experimental.pallas import tpu as pltpu
from jax.experimental.pallas import tpu_sc as plsc

_TAU = 0.5
_ALPHA = 0.5
_LAM = 0.5

_BN = 10000


def _tc_body(q_ref, hqb_ref, haqb_ref, h_ref, ha_ref, out_ref, acc_h, acc_a):
    g = pl.program_id(0)
    k = pl.num_programs(0)
    row = q_ref[0] % 8
    hq = hqb_ref[pl.ds(row, 1), :]
    haq = haqb_ref[pl.ds(row, 1), :]

    inv_aq = jnp.minimum(lax.rsqrt(jnp.sum(hq * hq)), 1e8)
    inv_aaq = jnp.minimum(lax.rsqrt(jnp.sum(haq * haq)), 1e8)
    ri = lax.broadcasted_iota(jnp.int32, (8, 1), 0)
    scale = jnp.where(ri == 0, inv_aq, jnp.where(ri == 1, inv_aaq, 0.0)) / _TAU

    w8 = jnp.concatenate([hq, haq, jnp.zeros((6, 128), jnp.float32)], axis=0)
    ones1 = jnp.ones((1, 128), jnp.float32)
    dn = (((1,), (1,)), ((), ()))

    @pl.when(g == 0)
    def _():
        acc_h[...] = jnp.zeros_like(acc_h)
        acc_a[...] = jnp.zeros_like(acc_a)

    def accum(blk, acc):
        d = lax.dot_general(w8, blk, dn, preferred_element_type=jnp.float32)
        nsq = lax.dot_general(ones1, blk * blk, dn,
                              preferred_element_type=jnp.float32)
        inv_n = jnp.minimum(lax.rsqrt(nsq), 1e8)
        e = jnp.exp(d * inv_n * scale)
        acc[...] += jnp.sum(e, axis=1, keepdims=True)

    accum(h_ref[...], acc_h)
    accum(ha_ref[...], acc_a)

    @pl.when(g == k - 1)
    def _():
        out_ref[:, 0:1] = acc_h[...]
        out_ref[:, 1:2] = acc_a[...]


def _tc_sums(h, h_aug, q1):
    n = h.shape[0]
    grid = n // _BN
    grid_spec = pltpu.PrefetchScalarGridSpec(
        num_scalar_prefetch=1,
        grid=(grid,),
        in_specs=[
            pl.BlockSpec((8, 128), lambda g, qr: (qr[0] // 8, 0)),
            pl.BlockSpec((8, 128), lambda g, qr: (qr[0] // 8, 0)),
            pl.BlockSpec((_BN, 128), lambda g, qr: (g, 0)),
            pl.BlockSpec((_BN, 128), lambda g, qr: (g, 0)),
        ],
        out_specs=pl.BlockSpec((8, 2), lambda g, qr: (0, 0)),
        scratch_shapes=[
            pltpu.VMEM((8, 1), jnp.float32),
            pltpu.VMEM((8, 1), jnp.float32),
        ],
    )
    return pl.pallas_call(
        _tc_body,
        grid_spec=grid_spec,
        out_shape=jax.ShapeDtypeStruct((8, 2), jnp.float32),
    )(q1, h, h_aug, h, h_aug)


def _sum16(x):
    li = lax.broadcasted_iota(jnp.int32, (16,), 0)
    for sh in (8, 4, 2, 1):
        x = x + jnp.take(x, jnp.bitwise_xor(li, sh))
    return x


def _nrsqrt16(x):
    x = jnp.maximum(x, 1e-30)
    i = lax.bitcast_convert_type(x, jnp.int32)
    i = 0x5F3759DF - lax.shift_right_arithmetic(i, 1)
    y = lax.bitcast_convert_type(i, jnp.float32)
    for _ in range(4):
        y = y * (1.5 - 0.5 * x * y * y)
    return jnp.minimum(y, 1e8)


def _sc_pos(h, h_aug, pos, q16):
    n = h.shape[0]
    p = pos.shape[0]
    nc, ns = 1, 16
    rpt = p // ns
    mesh = plsc.VectorSubcoreMesh(core_axis_name="c", subcore_axis_name="s",
                                  num_cores=nc, num_subcores=ns)

    @functools.partial(
        pl.kernel,
        out_type=jax.ShapeDtypeStruct((nc, 16), jnp.float32),
        mesh=mesh,
        scratch_types=[
            pltpu.VMEM((rpt,), jnp.int32),
            pltpu.VMEM((rpt,), jnp.int32),
            pltpu.VMEM((rpt, 128), jnp.float32),
            pltpu.VMEM((rpt, 128), jnp.float32),
            pltpu.VMEM((1, 128), jnp.float32),
            pltpu.VMEM((1, 128), jnp.float32),
            pltpu.VMEM((rpt,), jnp.int32),
            pltpu.VMEM((1, 16), jnp.float32),
            pltpu.VMEM((16, 16), jnp.float32),
            pltpu.VMEM((1, 16), jnp.int32),
            pltpu.VMEM_SHARED((n,), jnp.int32),
            pltpu.VMEM_SHARED((16, 16), jnp.float32),
            pltpu.SemaphoreType.DMA,
            pltpu.SemaphoreType.DMA,
        ],
    )
    def k(h_hbm, ha_hbm, pos_hbm, q_hbm, out_hbm,
          idx_v, gid_v, rows_h, rows_a, hq_v, haq_v, t_v, part_v, tab_v,
          q_v, winner_sh, tab_sh, sem, sem2):
        c = lax.axis_index("c")
        s = lax.axis_index("s")
        base = s * rpt
        li = lax.broadcasted_iota(jnp.int32, (16,), 0)

        pltpu.sync_copy(pos_hbm.at[pl.ds(base, rpt)], idx_v)
        for g2 in range(rpt // 16):
            gid_v[pl.ds(16 * g2, 16)] = li + (base + 16 * g2)

        cp_h = pltpu.async_copy(h_hbm.at[idx_v], rows_h, sem)
        cp_a = pltpu.async_copy(ha_hbm.at[idx_v], rows_a, sem2)

        pltpu.sync_copy(gid_v, winner_sh.at[idx_v])
        pltpu.sync_copy(q_hbm, q_v)
        qvec = q_v[0, :]
        qs = qvec[0]
        pltpu.sync_copy(h_hbm.at[pl.ds(qs, 1)], hq_v)
        pltpu.sync_copy(ha_hbm.at[pl.ds(qs, 1)], haq_v)
        plsc.subcore_barrier()
        pltpu.sync_copy(winner_sh.at[idx_v], t_v)
        cp_h.wait()
        cp_a.wait()

        accq = jnp.zeros((16,), jnp.float32)
        accqa = jnp.zeros((16,), jnp.float32)
        accx = jnp.zeros((16,), jnp.float32)
        for ch in range(8):
            a = hq_v[0, pl.ds(16 * ch, 16)]
            b = haq_v[0, pl.ds(16 * ch, 16)]
            accq += a * a
            accqa += b * b
            accx += a * b
        inv_aq = _nrsqrt16(_sum16(accq))
        inv_aaq = _nrsqrt16(_sum16(accqa))
        cq_v = _sum16(accx) * inv_aq * inv_aaq

        m1 = jnp.zeros((16,), jnp.float32)
        ma1 = jnp.zeros((16,), jnp.float32)
        m2 = jnp.zeros((16,), jnp.float32)
        ma2 = jnp.zeros((16,), jnp.float32)
        cntv = jnp.zeros((16,), jnp.float32)
        hqc = [hq_v[0, pl.ds(16 * ch, 16)] for ch in range(8)]
        haqc = [haq_v[0, pl.ds(16 * ch, 16)] for ch in range(8)]

        for g2 in range(rpt // 16):
            def row_body(i, carry, _g2=g2):
                d1v, da2v, nnv, d2v, da1v, nav = carry
                r = 16 * _g2 + i
                a1 = a2 = nn = b1 = b2 = bb = jnp.zeros((16,), jnp.float32)
                for ch in range(8):
                    xh = rows_h[r, pl.ds(16 * ch, 16)]
                    xa = rows_a[r, pl.ds(16 * ch, 16)]
                    a1 += xh * hqc[ch]
                    a2 += xh * haqc[ch]
                    nn += xh * xh
                    b1 += xa * hqc[ch]
                    b2 += xa * haqc[ch]
                    bb += xa * xa
                sel = li == i
                return (jnp.where(sel, _sum16(a1), d1v),
                        jnp.where(sel, _sum16(a2), da2v),
                        jnp.where(sel, _sum16(nn), nnv),
                        jnp.where(sel, _sum16(b1), d2v),
                        jnp.where(sel, _sum16(b2), da1v),
                        jnp.where(sel, _sum16(bb), nav))

            z6 = tuple(jnp.zeros((16,), jnp.float32) for _ in range(6))
            d1v, da2v, nnv, d2v, da1v, nav = lax.fori_loop(0, 16, row_body, z6)
            inv_h = _nrsqrt16(nnv)
            inv_a = _nrsqrt16(nav)
            tt = t_v[pl.ds(16 * g2, 16)]
            pv = idx_v[pl.ds(16 * g2, 16)]
            gids = li + 16 * g2 + base
            u = jnp.where((tt == gids) & (pv != qvec), 1.0, 0.0)
            m1 += u * (d1v * inv_h * inv_aq)
            ma2 += u * (da2v * inv_h * inv_aaq)
            m2 += u * (d2v * inv_a * inv_aq)
            ma1 += u * (da1v * inv_a * inv_aaq)
            cntv += u

        tile0 = jnp.where(s == 0, 1.0, 0.0)
        pvec = (jnp.where(li == 0, _sum16(m1), 0.0)
                + jnp.where(li == 1, _sum16(ma1), 0.0)
                + jnp.where(li == 2, _sum16(m2), 0.0)
                + jnp.where(li == 3, _sum16(ma2), 0.0)
                + jnp.where(li == 4, _sum16(cntv), 0.0)
                + jnp.where(li == 5, cq_v * tile0, 0.0))
        part_v[0, :] = pvec
        pltpu.sync_copy(part_v, tab_sh.at[pl.ds(s, 1)])
        plsc.subcore_barrier()

        @pl.when(s == 0)
        def _():
            pltpu.sync_copy(tab_sh, tab_v)
            acc = tab_v[0, :]
            for r in range(1, 16):
                acc = acc + tab_v[r, :]
            part_v[0, :] = acc
            pltpu.sync_copy(part_v, out_hbm.at[pl.ds(c, 1)])

    return k(h, h_aug, pos, q16)


def kernel(h, h_aug, q, pos, edge_index):
    n, d = h.shape
    qi = jnp.asarray(q, jnp.int32)
    q1 = jnp.full((1,), qi, jnp.int32)
    q16 = jnp.full((1, 16), qi, jnp.int32)

    sc = _sc_pos(h, h_aug, pos, q16)
    sums = _tc_sums(h, h_aug, q1)

    li = jnp.arange(16)
    wm = (0.5 * ((li == 0) | (li == 1)) + 0.25 * ((li == 2) | (li == 3))
          ).astype(jnp.float32)
    r = sc[0]
    dotm = jnp.sum(r * wm)
    cnt = jnp.sum(r * (li == 4))
    cq = jnp.sum(r * (li == 5))
    log_term = 0.5 * jnp.sum(jnp.log(sums[0:2, :]))
    return log_term - dotm / (_TAU * cnt) - 0.5 * cq / _TAU
